# baseline (speedup 1.0000x reference)
import functools

import jax
import jax.numpy as jnp
from jax import lax
from jax.experimental import pallas as pl
from jax.experimental.pallas import tpu as pltpu
from jax.experimental.pallas import tpu_sc as plsc

N = 10000
E = 320000
CUR = 128
HID = 128
OUT = 128
HEADS = 6

NC = 2
NS = 16
LANES = 16
NW = NC * NS
EPW = E // NW
CHUNK = 16
NCHUNK = EPW // CHUNK
TAIL = EPW - NCHUNK * CHUNK
RPW = N // NS
TAB = 80
IDEP = 8
GDEP = 4

BLK = 1000


def _pre_body(x_ref, wp_ref, bp_ref, wg_ref, a8s_ref, a8d_ref,
              h_ref, hw6_ref, ast_ref, adt_ref):
    xb = x_ref[...]
    h = jnp.dot(xb, wp_ref[...], preferred_element_type=jnp.float32) + bp_ref[...]
    h_ref[...] = h
    hws = []
    for hh in range(HEADS):
        hws.append(jnp.dot(h, wg_ref[:, hh * OUT:(hh + 1) * OUT],
                           preferred_element_type=jnp.float32))
    hw6_ref[...] = jnp.stack(hws, axis=0)
    hwflat = jnp.concatenate(hws, axis=1)
    dn = (((1,), (1,)), ((), ()))
    ast_ref[...] = lax.dot_general(hwflat, a8s_ref[...], dn,
                                   preferred_element_type=jnp.float32)
    adt_ref[...] = lax.dot_general(hwflat, a8d_ref[...], dn,
                                   preferred_element_type=jnp.float32)


def _post_body(aggp_ref, denp_ref, hw6_ref, ast_ref, adt_ref, h_ref,
               wl_ref, bl_ref, bg_ref, out_ref):
    hb = h_ref[...]
    acc = jnp.dot(hb, wl_ref[:HID, :], preferred_element_type=jnp.float32)
    for hh in range(HEADS):
        a_self = ast_ref[:, hh] + adt_ref[:, hh]
        ws = jnp.exp(jnp.maximum(a_self, 0.2 * a_self))
        hwh = hw6_ref[hh]
        agg = (aggp_ref[0, hh] + aggp_ref[1, hh]
               + ws[:, None] * hwh)
        den = jnp.sum(denp_ref[hh], axis=1) + ws
        h1 = agg / den[:, None] + bg_ref[hh][None, :]
        h1 = jnp.maximum(h1, 0.01 * h1)
        acc = acc + jnp.dot(h1, wl_ref[HID + hh * OUT:HID + (hh + 1) * OUT, :],
                            preferred_element_type=jnp.float32)
    out_ref[...] = acc + bl_ref[...]


def _sc_body(hw6_hbm, ast_hbm, adt_hbm, src_hbm, dst_hbm,
             agg_out, den_out,
             shared_agg, s_t, d_t, gbuf, wbuf, srcc, dstc,
             denom_t, stail, dtail, sem_g, sem_i, sem_s):
    ci = lax.axis_index("c")
    si = lax.axis_index("s")
    wid = ci * NS + si
    eb = wid * EPW

    own_off = pl.multiple_of(si * RPW - lax.rem(si, 8), 8)
    OWN = 632

    zero16 = jnp.zeros((LANES,), jnp.float32)

    def _idx_copy(tab_hbm, c, buf):
        return pltpu.make_async_copy(
            tab_hbm.at[pl.ds(eb + c * CHUNK, CHUNK)], buf, sem_i)

    def _wcompute(sv, dv):
        s = plsc.load_gather(
            s_t, [lax.shift_right_logical(sv, 7), lax.bitwise_and(sv, 127)])
        dd = plsc.load_gather(
            d_t, [lax.shift_right_logical(dv, 7), lax.bitwise_and(dv, 127)])
        a = s + dd
        a = jnp.maximum(a, 0.2 * a)
        w = jnp.exp(a)
        plsc.addupdate_scatter(
            denom_t, [lax.shift_right_logical(dv, 7),
                      lax.bitwise_and(dv, 127)], w)
        return w

    def _scale_rows(gslot, n, wslot):
        @plsc.parallel_loop(0, n, 1, unroll=8)
        def _scale(e):
            esplat = jnp.zeros((LANES,), jnp.int32) + e
            wv = plsc.load_gather(wslot, [esplat])
            row = gslot.at[e]
            for r in range(OUT // LANES):
                sl = pl.ds(r * LANES, LANES)
                row[sl] = row[sl] * wv

    def _wchunk(islot, wslot):
        for j in range(CHUNK // LANES):
            sl16 = pl.ds(j * LANES, LANES)
            wslot[sl16] = _wcompute(srcc.at[islot][sl16],
                                    dstc.at[islot][sl16])

    def _gather(hh, islot, gslot):
        return pltpu.make_async_copy(
            hw6_hbm.at[hh].at[srcc.at[islot]], gbuf.at[gslot], sem_g)

    def _scat(islot, gslot):
        return (gbuf.at[gslot], shared_agg.at[dstc.at[islot]])

    for hh in range(HEADS):
        def _zg(i, carry):
            for r in range(OUT // LANES):
                denom_t.at[i][pl.ds(r * LANES, LANES)] = zero16
            return carry
        lax.fori_loop(0, TAB, _zg, 0)

        def _zb(i, carry):
            for r in range(OUT // LANES):
                gbuf.at[0].at[i][pl.ds(r * LANES, LANES)] = zero16
            return carry
        lax.fori_loop(0, CHUNK, _zb, 0)

        for k in range(OWN // CHUNK):
            pltpu.sync_copy(gbuf.at[0],
                            shared_agg.at[pl.ds(own_off + k * CHUNK, CHUNK)])
        _zrem = OWN - (OWN // CHUNK) * CHUNK
        if _zrem:
            pltpu.sync_copy(gbuf.at[0].at[pl.ds(0, _zrem)],
                            shared_agg.at[pl.ds(own_off + OWN - _zrem,
                                                _zrem)])

        pltpu.sync_copy(ast_hbm.at[hh], s_t)
        pltpu.sync_copy(adt_hbm.at[hh], d_t)
        plsc.subcore_barrier()

        for k in range(3):
            _idx_copy(src_hbm, k, srcc.at[k]).start()
            _idx_copy(dst_hbm, k, dstc.at[k]).start()
        _idx_copy(src_hbm, 0, srcc.at[0]).wait()
        _idx_copy(dst_hbm, 0, dstc.at[0]).wait()
        _gather(hh, 0, 0).start()
        _wchunk(0, wbuf.at[0])

        def _chunk(c, carry):
            b3 = lax.rem(c, GDEP)
            nb3 = lax.rem(c + 1, GDEP)
            b8 = lax.rem(c, IDEP)
            nb8 = lax.rem(c + 1, IDEP)
            b2 = lax.rem(c, 2)
            nb2 = lax.rem(c + 1, 2)

            @pl.when(c >= GDEP - 1)
            def _():
                g, s = _scat(lax.rem(c - (GDEP - 1), IDEP), nb3)
                pltpu.make_async_copy(g, s, sem_s).wait()

            _idx_copy(src_hbm, c + 1, srcc.at[nb8]).wait()
            _idx_copy(dst_hbm, c + 1, dstc.at[nb8]).wait()

            @pl.when(c + 3 < NCHUNK)
            def _():
                pb8 = lax.rem(c + 3, IDEP)
                _idx_copy(src_hbm, c + 3, srcc.at[pb8]).start()
                _idx_copy(dst_hbm, c + 3, dstc.at[pb8]).start()

            _gather(hh, nb8, nb3).start()
            _wchunk(nb8, wbuf.at[nb2])

            _gather(hh, b8, b3).wait()
            _scale_rows(gbuf.at[b3], CHUNK, wbuf.at[b2])
            g, s = _scat(b8, b3)
            pltpu.async_copy(g, s, sem_s, add=True)
            return carry
        lax.fori_loop(0, NCHUNK - 1, _chunk, 0)

        F = NCHUNK - 1
        _gather(hh, F % IDEP, F % GDEP).wait()
        _scale_rows(gbuf.at[F % GDEP], CHUNK, wbuf.at[F % 2])
        g, s = _scat(F % IDEP, F % GDEP)
        pltpu.sync_copy(g, s, add=True)
        for cc in range(F - (GDEP - 1), F):
            g, s = _scat(cc % IDEP, cc % GDEP)
            pltpu.make_async_copy(g, s, sem_s).wait()

        tb = NCHUNK * CHUNK
        if TAIL:
            pltpu.make_async_copy(
                src_hbm.at[pl.ds(eb + tb, TAIL)], stail, sem_i).start()
            pltpu.make_async_copy(
                dst_hbm.at[pl.ds(eb + tb, TAIL)], dtail, sem_i).start()
            pltpu.make_async_copy(
                src_hbm.at[pl.ds(eb + tb, TAIL)], stail, sem_i).wait()
            pltpu.make_async_copy(
                dst_hbm.at[pl.ds(eb + tb, TAIL)], dtail, sem_i).wait()
            gt = gbuf.at[0].at[pl.ds(0, TAIL)]
            cp = pltpu.make_async_copy(hw6_hbm.at[hh].at[stail], gt, sem_g)
            cp.start()
            w = _wcompute(stail[...], dtail[...])
            wbuf.at[0][pl.ds(0, TAIL)] = w
            cp.wait()
            _scale_rows(gbuf.at[0], TAIL, wbuf.at[0])
            pltpu.sync_copy(gt, shared_agg.at[dtail], add=True)

        plsc.subcore_barrier()

        pltpu.sync_copy(shared_agg.at[pl.ds(own_off, OWN)],
                        agg_out.at[ci, hh, pl.ds(own_off, OWN)])
        pltpu.sync_copy(denom_t, den_out.at[hh, wid])


def _sc_call(hw6, ast, adt, src, dst):
    mesh = plsc.VectorSubcoreMesh(core_axis_name="c", subcore_axis_name="s",
                                  num_cores=NC, num_subcores=NS)
    f = pl.kernel(
        _sc_body,
        out_type=[jax.ShapeDtypeStruct((NC, HEADS, N, OUT), jnp.float32),
                  jax.ShapeDtypeStruct((HEADS, NW, TAB, OUT), jnp.float32)],
        mesh=mesh,
        compiler_params=pltpu.CompilerParams(needs_layout_passes=False),
        scratch_types=[
            pltpu.VMEM_SHARED((N, OUT), jnp.float32),
            pltpu.VMEM((TAB, OUT), jnp.float32),
            pltpu.VMEM((TAB, OUT), jnp.float32),
            pltpu.VMEM((GDEP, CHUNK, OUT), jnp.float32),
            pltpu.VMEM((2, CHUNK), jnp.float32),
            pltpu.VMEM((IDEP, CHUNK), jnp.int32),
            pltpu.VMEM((IDEP, CHUNK), jnp.int32),
            pltpu.VMEM((TAB, OUT), jnp.float32),
            pltpu.VMEM((max(TAIL, 16),), jnp.int32),
            pltpu.VMEM((max(TAIL, 16),), jnp.int32),
            pltpu.SemaphoreType.DMA,
            pltpu.SemaphoreType.DMA,
            pltpu.SemaphoreType.DMA,
        ],
    )
    return f(hw6, ast, adt, src, dst)


def kernel(x, edge_index, edge_weight, Wp, bp, Wg, att_src, att_dst, bg, Wl, bl):
    del edge_weight
    src = edge_index[0]
    dst = edge_index[1]

    eye = jnp.eye(HEADS, dtype=jnp.float32)
    a6s = (eye[:, :, None] * att_src[None, :, :]).reshape(HEADS, HEADS * OUT)
    a6d = (eye[:, :, None] * att_dst[None, :, :]).reshape(HEADS, HEADS * OUT)
    a8s = jnp.concatenate([a6s, jnp.zeros((2, HEADS * OUT), jnp.float32)], axis=0)
    a8d = jnp.concatenate([a6d, jnp.zeros((2, HEADS * OUT), jnp.float32)], axis=0)

    grid = N // BLK
    h, hw6, ast, adt = pl.pallas_call(
        _pre_body,
        grid=(grid,),
        in_specs=[
            pl.BlockSpec((BLK, CUR), lambda i: (i, 0)),
            pl.BlockSpec((CUR, HID), lambda i: (0, 0)),
            pl.BlockSpec((1, HID), lambda i: (0, 0)),
            pl.BlockSpec((HID, HEADS * OUT), lambda i: (0, 0)),
            pl.BlockSpec((8, HEADS * OUT), lambda i: (0, 0)),
            pl.BlockSpec((8, HEADS * OUT), lambda i: (0, 0)),
        ],
        out_specs=[
            pl.BlockSpec((BLK, HID), lambda i: (i, 0)),
            pl.BlockSpec((HEADS, BLK, OUT), lambda i: (0, i, 0)),
            pl.BlockSpec((BLK, 8), lambda i: (i, 0)),
            pl.BlockSpec((BLK, 8), lambda i: (i, 0)),
        ],
        out_shape=[
            jax.ShapeDtypeStruct((N, HID), jnp.float32),
            jax.ShapeDtypeStruct((HEADS, N, OUT), jnp.float32),
            jax.ShapeDtypeStruct((N, 8), jnp.float32),
            jax.ShapeDtypeStruct((N, 8), jnp.float32),
        ],
    )(x, Wp, bp.reshape(1, HID), Wg, a8s, a8d)

    pad = TAB * OUT - N
    astT = jnp.pad(ast.T, ((0, 0), (0, pad))).reshape(8, TAB, OUT)
    adtT = jnp.pad(adt.T, ((0, 0), (0, pad))).reshape(8, TAB, OUT)
    aggp, denp = _sc_call(hw6, astT, adtT, src, dst)
    denp = denp.reshape(HEADS, NW, TAB * OUT)[:, :, :N].transpose(0, 2, 1)

    out = pl.pallas_call(
        _post_body,
        grid=(grid,),
        in_specs=[
            pl.BlockSpec((NC, HEADS, BLK, OUT), lambda i: (0, 0, i, 0)),
            pl.BlockSpec((HEADS, BLK, NW), lambda i: (0, i, 0)),
            pl.BlockSpec((HEADS, BLK, OUT), lambda i: (0, i, 0)),
            pl.BlockSpec((BLK, 8), lambda i: (i, 0)),
            pl.BlockSpec((BLK, 8), lambda i: (i, 0)),
            pl.BlockSpec((BLK, HID), lambda i: (i, 0)),
            pl.BlockSpec((HID + HEADS * OUT, OUT), lambda i: (0, 0)),
            pl.BlockSpec((1, OUT), lambda i: (0, 0)),
            pl.BlockSpec((HEADS, OUT), lambda i: (0, 0)),
        ],
        out_specs=pl.BlockSpec((BLK, OUT), lambda i: (i, 0)),
        out_shape=jax.ShapeDtypeStruct((N, OUT), jnp.float32),
    )(aggp, denp, hw6, ast, adt, h, Wl, bl.reshape(1, OUT), bg.reshape(HEADS, OUT))

    return out


# speedup vs baseline: 1.8041x; 1.8041x over previous
import functools

import jax
import jax.numpy as jnp
from jax import lax
from jax.experimental import pallas as pl
from jax.experimental.pallas import tpu as pltpu
from jax.experimental.pallas import tpu_sc as plsc

N = 10000
E = 320000
CUR = 128
HID = 128
OUT = 128
HEADS = 6

NC = 2
NS = 16
LANES = 16
NW = NC * NS
EPW = E // NW
CHUNK = 64
NCHUNK = EPW // CHUNK
TAIL = EPW - NCHUNK * CHUNK
RPW = N // NS
TAB = 80
IDEP = 6
GDEP = 3

BLK = 1000


def _pre_body(x_ref, wp_ref, bp_ref, wg_ref, a8s_ref, a8d_ref,
              h_ref, hw6_ref, ast_ref, adt_ref):
    xb = x_ref[...]
    h = jnp.dot(xb, wp_ref[...], preferred_element_type=jnp.float32) + bp_ref[...]
    h_ref[...] = h
    hws = []
    for hh in range(HEADS):
        hws.append(jnp.dot(h, wg_ref[:, hh * OUT:(hh + 1) * OUT],
                           preferred_element_type=jnp.float32))
    hw6_ref[...] = jnp.stack(hws, axis=0)
    hwflat = jnp.concatenate(hws, axis=1)
    dn = (((1,), (1,)), ((), ()))
    ast_ref[...] = lax.dot_general(hwflat, a8s_ref[...], dn,
                                   preferred_element_type=jnp.float32)
    adt_ref[...] = lax.dot_general(hwflat, a8d_ref[...], dn,
                                   preferred_element_type=jnp.float32)


def _post_body(aggp_ref, denp_ref, hw6_ref, ast_ref, adt_ref, h_ref,
               wl_ref, bl_ref, bg_ref, out_ref):
    hb = h_ref[...]
    acc = jnp.dot(hb, wl_ref[:HID, :], preferred_element_type=jnp.float32)
    for hh in range(HEADS):
        a_self = ast_ref[:, hh] + adt_ref[:, hh]
        ws = jnp.exp(jnp.maximum(a_self, 0.2 * a_self))
        hwh = hw6_ref[hh]
        agg = (aggp_ref[0, hh] + aggp_ref[1, hh]
               + ws[:, None] * hwh)
        den = jnp.sum(denp_ref[hh], axis=1) + ws
        h1 = agg / den[:, None] + bg_ref[hh][None, :]
        h1 = jnp.maximum(h1, 0.01 * h1)
        acc = acc + jnp.dot(h1, wl_ref[HID + hh * OUT:HID + (hh + 1) * OUT, :],
                            preferred_element_type=jnp.float32)
    out_ref[...] = acc + bl_ref[...]


def _sc_body(hw6_hbm, sd_hbm, src_hbm, dst_hbm,
             agg_out, den_out,
             shared_agg, sd_t, gbuf, wbuf, srcc, dstc,
             denom_t, stail, dtail, sem_g, sem_i, sem_s):
    ci = lax.axis_index("c")
    si = lax.axis_index("s")
    wid = ci * NS + si
    eb = wid * EPW

    own_off = pl.multiple_of(si * RPW - lax.rem(si, 8), 8)
    OWN = 632

    zero16 = jnp.zeros((LANES,), jnp.float32)

    def _idx_copy(tab_hbm, c, buf):
        return pltpu.make_async_copy(
            tab_hbm.at[pl.ds(eb + c * CHUNK, CHUNK)], buf, sem_i)

    def _wcompute(sv, dv):
        ws_word = plsc.load_gather(
            sd_t, [lax.shift_right_logical(sv, 7), lax.bitwise_and(sv, 127)])
        wd_word = plsc.load_gather(
            sd_t, [lax.shift_right_logical(dv, 7), lax.bitwise_and(dv, 127)])
        s_i = lax.shift_right_arithmetic(lax.shift_left(ws_word, 16), 16)
        d_i = lax.shift_right_arithmetic(wd_word, 16)
        a = lax.convert_element_type(s_i + d_i, jnp.float32) * (1.0 / 2048.0)
        a = jnp.maximum(a, 0.2 * a)
        w = jnp.exp(a)
        plsc.addupdate_scatter(
            denom_t, [lax.shift_right_logical(dv, 7),
                      lax.bitwise_and(dv, 127)], w)
        return w

    def _scale_rows(gslot, n, wslot):
        @plsc.parallel_loop(0, n, 1, unroll=8)
        def _scale(e):
            esplat = jnp.zeros((LANES,), jnp.int32) + e
            wv = plsc.load_gather(wslot, [esplat])
            row = gslot.at[e]
            for r in range(OUT // LANES):
                sl = pl.ds(r * LANES, LANES)
                row[sl] = row[sl] * wv

    def _wchunk(islot, wslot):
        for j in range(CHUNK // LANES):
            sl16 = pl.ds(j * LANES, LANES)
            wslot[sl16] = _wcompute(srcc.at[islot][sl16],
                                    dstc.at[islot][sl16])

    def _gather(hh, islot, gslot):
        return pltpu.make_async_copy(
            hw6_hbm.at[hh].at[srcc.at[islot]], gbuf.at[gslot], sem_g)

    def _scat(islot, gslot):
        return (gbuf.at[gslot], shared_agg.at[dstc.at[islot]])

    for hh in range(HEADS):
        def _zg(i, carry):
            for r in range(OUT // LANES):
                denom_t.at[i][pl.ds(r * LANES, LANES)] = zero16
            return carry
        lax.fori_loop(0, TAB, _zg, 0)

        def _zb(i, carry):
            for r in range(OUT // LANES):
                gbuf.at[0].at[i][pl.ds(r * LANES, LANES)] = zero16
            return carry
        lax.fori_loop(0, CHUNK, _zb, 0)

        for k in range(OWN // CHUNK):
            pltpu.sync_copy(gbuf.at[0],
                            shared_agg.at[pl.ds(own_off + k * CHUNK, CHUNK)])
        _zrem = OWN - (OWN // CHUNK) * CHUNK
        if _zrem:
            pltpu.sync_copy(gbuf.at[0].at[pl.ds(0, _zrem)],
                            shared_agg.at[pl.ds(own_off + OWN - _zrem,
                                                _zrem)])

        pltpu.sync_copy(sd_hbm.at[hh], sd_t)
        plsc.subcore_barrier()

        for k in range(3):
            _idx_copy(src_hbm, k, srcc.at[k]).start()
            _idx_copy(dst_hbm, k, dstc.at[k]).start()
        _idx_copy(src_hbm, 0, srcc.at[0]).wait()
        _idx_copy(dst_hbm, 0, dstc.at[0]).wait()
        _gather(hh, 0, 0).start()
        _wchunk(0, wbuf.at[0])

        def _chunk(c, carry):
            b3 = lax.rem(c, GDEP)
            nb3 = lax.rem(c + 1, GDEP)
            b8 = lax.rem(c, IDEP)
            nb8 = lax.rem(c + 1, IDEP)
            b2 = lax.rem(c, 2)
            nb2 = lax.rem(c + 1, 2)

            @pl.when(c >= GDEP - 1)
            def _():
                g, s = _scat(lax.rem(c - (GDEP - 1), IDEP), nb3)
                pltpu.make_async_copy(g, s, sem_s).wait()

            _idx_copy(src_hbm, c + 1, srcc.at[nb8]).wait()
            _idx_copy(dst_hbm, c + 1, dstc.at[nb8]).wait()

            @pl.when(c + 3 < NCHUNK)
            def _():
                pb8 = lax.rem(c + 3, IDEP)
                _idx_copy(src_hbm, c + 3, srcc.at[pb8]).start()
                _idx_copy(dst_hbm, c + 3, dstc.at[pb8]).start()

            _gather(hh, nb8, nb3).start()
            _wchunk(nb8, wbuf.at[nb2])

            _gather(hh, b8, b3).wait()
            _scale_rows(gbuf.at[b3], CHUNK, wbuf.at[b2])
            g, s = _scat(b8, b3)
            pltpu.async_copy(g, s, sem_s, add=True)
            return carry
        lax.fori_loop(0, NCHUNK - 1, _chunk, 0)

        F = NCHUNK - 1
        _gather(hh, F % IDEP, F % GDEP).wait()
        _scale_rows(gbuf.at[F % GDEP], CHUNK, wbuf.at[F % 2])
        g, s = _scat(F % IDEP, F % GDEP)
        pltpu.sync_copy(g, s, add=True)
        for cc in range(F - (GDEP - 1), F):
            g, s = _scat(cc % IDEP, cc % GDEP)
            pltpu.make_async_copy(g, s, sem_s).wait()

        tb = NCHUNK * CHUNK
        if TAIL:
            pltpu.make_async_copy(
                src_hbm.at[pl.ds(eb + tb, TAIL)], stail, sem_i).start()
            pltpu.make_async_copy(
                dst_hbm.at[pl.ds(eb + tb, TAIL)], dtail, sem_i).start()
            pltpu.make_async_copy(
                src_hbm.at[pl.ds(eb + tb, TAIL)], stail, sem_i).wait()
            pltpu.make_async_copy(
                dst_hbm.at[pl.ds(eb + tb, TAIL)], dtail, sem_i).wait()
            gt = gbuf.at[0].at[pl.ds(0, TAIL)]
            cp = pltpu.make_async_copy(hw6_hbm.at[hh].at[stail], gt, sem_g)
            cp.start()
            w = _wcompute(stail[...], dtail[...])
            wbuf.at[0][pl.ds(0, TAIL)] = w
            cp.wait()
            _scale_rows(gbuf.at[0], TAIL, wbuf.at[0])
            pltpu.sync_copy(gt, shared_agg.at[dtail], add=True)

        plsc.subcore_barrier()

        pltpu.sync_copy(shared_agg.at[pl.ds(own_off, OWN)],
                        agg_out.at[ci, hh, pl.ds(own_off, OWN)])
        pltpu.sync_copy(denom_t, den_out.at[hh, wid])


def _sc_call(hw6, sd, src, dst):
    mesh = plsc.VectorSubcoreMesh(core_axis_name="c", subcore_axis_name="s",
                                  num_cores=NC, num_subcores=NS)
    f = pl.kernel(
        _sc_body,
        out_type=[jax.ShapeDtypeStruct((NC, HEADS, N, OUT), jnp.float32),
                  jax.ShapeDtypeStruct((HEADS, NW, TAB, OUT), jnp.float32)],
        mesh=mesh,
        compiler_params=pltpu.CompilerParams(needs_layout_passes=False),
        scratch_types=[
            pltpu.VMEM_SHARED((N, OUT), jnp.float32),
            pltpu.VMEM((TAB, OUT), jnp.int32),
            pltpu.VMEM((GDEP, CHUNK, OUT), jnp.float32),
            pltpu.VMEM((2, CHUNK), jnp.float32),
            pltpu.VMEM((IDEP, CHUNK), jnp.int32),
            pltpu.VMEM((IDEP, CHUNK), jnp.int32),
            pltpu.VMEM((TAB, OUT), jnp.float32),
            pltpu.VMEM((max(TAIL, 16),), jnp.int32),
            pltpu.VMEM((max(TAIL, 16),), jnp.int32),
            pltpu.SemaphoreType.DMA,
            pltpu.SemaphoreType.DMA,
            pltpu.SemaphoreType.DMA,
        ],
    )
    return f(hw6, sd, src, dst)


def kernel(x, edge_index, edge_weight, Wp, bp, Wg, att_src, att_dst, bg, Wl, bl):
    del edge_weight
    src = edge_index[0]
    dst = edge_index[1]

    eye = jnp.eye(HEADS, dtype=jnp.float32)
    a6s = (eye[:, :, None] * att_src[None, :, :]).reshape(HEADS, HEADS * OUT)
    a6d = (eye[:, :, None] * att_dst[None, :, :]).reshape(HEADS, HEADS * OUT)
    a8s = jnp.concatenate([a6s, jnp.zeros((2, HEADS * OUT), jnp.float32)], axis=0)
    a8d = jnp.concatenate([a6d, jnp.zeros((2, HEADS * OUT), jnp.float32)], axis=0)

    grid = N // BLK
    h, hw6, ast, adt = pl.pallas_call(
        _pre_body,
        grid=(grid,),
        in_specs=[
            pl.BlockSpec((BLK, CUR), lambda i: (i, 0)),
            pl.BlockSpec((CUR, HID), lambda i: (0, 0)),
            pl.BlockSpec((1, HID), lambda i: (0, 0)),
            pl.BlockSpec((HID, HEADS * OUT), lambda i: (0, 0)),
            pl.BlockSpec((8, HEADS * OUT), lambda i: (0, 0)),
            pl.BlockSpec((8, HEADS * OUT), lambda i: (0, 0)),
        ],
        out_specs=[
            pl.BlockSpec((BLK, HID), lambda i: (i, 0)),
            pl.BlockSpec((HEADS, BLK, OUT), lambda i: (0, i, 0)),
            pl.BlockSpec((BLK, 8), lambda i: (i, 0)),
            pl.BlockSpec((BLK, 8), lambda i: (i, 0)),
        ],
        out_shape=[
            jax.ShapeDtypeStruct((N, HID), jnp.float32),
            jax.ShapeDtypeStruct((HEADS, N, OUT), jnp.float32),
            jax.ShapeDtypeStruct((N, 8), jnp.float32),
            jax.ShapeDtypeStruct((N, 8), jnp.float32),
        ],
    )(x, Wp, bp.reshape(1, HID), Wg, a8s, a8d)

    pad = TAB * OUT - N
    astT = jnp.pad(ast.T, ((0, 0), (0, pad)))
    adtT = jnp.pad(adt.T, ((0, 0), (0, pad)))
    sb = jnp.clip(jnp.round(astT * 2048.0), -32768, 32767).astype(jnp.int32)
    db = jnp.clip(jnp.round(adtT * 2048.0), -32768, 32767).astype(jnp.int32)
    sd = ((sb & 0xFFFF) | (db << 16)).astype(jnp.int32).reshape(8, TAB, OUT)
    aggp, denp = _sc_call(hw6, sd, src, dst)
    denp = denp.reshape(HEADS, NW, TAB * OUT)[:, :, :N].transpose(0, 2, 1)

    out = pl.pallas_call(
        _post_body,
        grid=(grid,),
        in_specs=[
            pl.BlockSpec((NC, HEADS, BLK, OUT), lambda i: (0, 0, i, 0)),
            pl.BlockSpec((HEADS, BLK, NW), lambda i: (0, i, 0)),
            pl.BlockSpec((HEADS, BLK, OUT), lambda i: (0, i, 0)),
            pl.BlockSpec((BLK, 8), lambda i: (i, 0)),
            pl.BlockSpec((BLK, 8), lambda i: (i, 0)),
            pl.BlockSpec((BLK, HID), lambda i: (i, 0)),
            pl.BlockSpec((HID + HEADS * OUT, OUT), lambda i: (0, 0)),
            pl.BlockSpec((1, OUT), lambda i: (0, 0)),
            pl.BlockSpec((HEADS, OUT), lambda i: (0, 0)),
        ],
        out_specs=pl.BlockSpec((BLK, OUT), lambda i: (i, 0)),
        out_shape=jax.ShapeDtypeStruct((N, OUT), jnp.float32),
    )(aggp, denp, hw6, ast, adt, h, Wl, bl.reshape(1, OUT), bg.reshape(HEADS, OUT))

    return out
